# baseline (speedup 1.0000x reference)
import functools
import math

import jax
import jax.numpy as jnp
from jax import lax
from jax.experimental import pallas as pl
from jax.experimental.pallas import tpu as pltpu
from jax.experimental.pallas import tpu_sc as plsc

_CUTOFF = 4.0
_NUM_BASIS = 16

_NC = 2
_NS = 16
_NW = _NC * _NS

_C = 2000


def _sc_edge_diff(pos_flat, ei_flat, n):
    e = ei_flat.shape[0] // 2
    assert e % _NW == 0
    ew = e // _NW
    assert ew % _C == 0

    mesh = plsc.VectorSubcoreMesh(core_axis_name="c", subcore_axis_name="s")
    out_type = (
        jax.ShapeDtypeStruct((e,), jnp.float32),
        jax.ShapeDtypeStruct((e,), jnp.float32),
        jax.ShapeDtypeStruct((e,), jnp.float32),
    )
    scratch = [
        pltpu.VMEM((n,), jnp.float32),
        pltpu.VMEM((_C,), jnp.int32),
        pltpu.VMEM((_C,), jnp.int32),
        pltpu.VMEM((_C,), jnp.float32),
    ]

    @functools.partial(
        pl.kernel, out_type=out_type, mesh=mesh, scratch_types=scratch,
        compiler_params=pltpu.CompilerParams(needs_layout_passes=False))
    def sc_kernel(pos_hbm, ei_hbm, d0_hbm, d1_hbm, d2_hbm,
                  plane, sidx, didx, dbuf):
        wid = lax.axis_index("s") * _NC + lax.axis_index("c")

        for pofs, d_hbm in ((0, d0_hbm), (n, d1_hbm), (2 * n, d2_hbm)):
            pltpu.sync_copy(pos_hbm.at[pl.ds(pofs, n)], plane)

            def chunk(kk, _):
                cb = wid * ew + kk * _C
                pltpu.sync_copy(ei_hbm.at[pl.ds(cb, _C)], sidx)
                pltpu.sync_copy(ei_hbm.at[pl.ds(e + cb, _C)], didx)

                def step(i, _):
                    sl = pl.ds(i * 16, 16)
                    a = plsc.load_gather(plane, [didx[sl]])
                    b = plsc.load_gather(plane, [sidx[sl]])
                    dbuf[sl] = a - b
                    return 0

                lax.fori_loop(0, _C // 16, step, 0)
                pltpu.sync_copy(dbuf, d_hbm.at[pl.ds(cb, _C)])
                return 0

            lax.fori_loop(0, ew // _C, chunk, 0)

    return sc_kernel(pos_flat, ei_flat)


def _tc_node_embed(x, emb2t):
    n = x.shape[0]
    b = 3200
    assert n % b == 0
    nb = n // b
    xr = x.reshape(nb, 1, b)
    s = emb2t.shape[1]

    def body(x_ref, e_ref, hx_ref, hz_ref):
        xv = x_ref[...].reshape(1, b)
        sp = lax.broadcasted_iota(jnp.int32, (s, 1), 0)
        onehot = (xv == sp).astype(jnp.float32)
        t = lax.dot_general(e_ref[...], onehot, (((1,), (0,)), ((), ())),
                            preferred_element_type=jnp.float32)
        hx_ref[...] = t[:8, :]
        hz_ref[...] = t[8:, :]

    return pl.pallas_call(
        body,
        grid=(nb,),
        in_specs=[pl.BlockSpec((1, 1, b), lambda i: (i, 0, 0)),
                  pl.BlockSpec((16, s), lambda i: (0, 0))],
        out_specs=[pl.BlockSpec((8, b), lambda i: (0, i)),
                   pl.BlockSpec((8, b), lambda i: (0, i))],
        out_shape=[jax.ShapeDtypeStruct((8, n), jnp.float32),
                   jax.ShapeDtypeStruct((8, n), jnp.float32)],
    )(xr, emb2t)


def _tc_bessel(d0, d1, d2):
    e = d0.shape[0]
    b = 3200
    assert e % b == 0
    nb = e // b
    dr = [d.reshape(nb, 1, b) for d in (d0, d1, d2)]

    def body(xr, yr, zr, h_ref, a_ref):
        dx = xr[...].reshape(1, b)
        dy = yr[...].reshape(1, b)
        dz = zr[...].reshape(1, b)
        r = jnp.sqrt(dx * dx + dy * dy + dz * dz)
        g = math.sqrt(2.0 / _CUTOFF) / r
        nvec = (lax.broadcasted_iota(jnp.int32, (_NUM_BASIS, 1), 0) + 1
                ).astype(jnp.float32)
        theta = jnp.broadcast_to(r * (math.pi / _CUTOFF), (_NUM_BASIS, b))
        h_ref[...] = jnp.sin(theta * nvec) * g
        a_ref[...] = jnp.concatenate([dx, dy, dz], axis=0)

    return pl.pallas_call(
        body,
        grid=(nb,),
        in_specs=[pl.BlockSpec((1, 1, b), lambda i: (i, 0, 0))] * 3,
        out_specs=[pl.BlockSpec((_NUM_BASIS, b), lambda i: (0, i)),
                   pl.BlockSpec((3, b), lambda i: (0, i))],
        out_shape=[jax.ShapeDtypeStruct((_NUM_BASIS, e), jnp.float32),
                   jax.ShapeDtypeStruct((3, e), jnp.float32)],
    )(*dr)


def kernel(x, positions, edge_index, embed_node_x, embed_node_z):
    n = positions.shape[0]
    pos_flat = positions.T.reshape(-1)
    ei_flat = edge_index.astype(jnp.int32).reshape(-1)
    emb2t = jnp.concatenate([embed_node_x, embed_node_z], axis=1).T
    npad = -n % 3200
    xpad = jnp.pad(x.astype(jnp.int32), (0, npad))
    hx, hz = _tc_node_embed(xpad, emb2t)
    hx = lax.slice(hx, (0, 0), (8, n))
    hz = lax.slice(hz, (0, 0), (8, n))
    d0, d1, d2 = _sc_edge_diff(pos_flat, ei_flat, n)
    h16, a3 = _tc_bessel(d0, d1, d2)
    return (hx.T, hz.T, h16.T, a3.T)


# speedup vs baseline: 1.1709x; 1.1709x over previous
import functools
import math

import jax
import jax.numpy as jnp
from jax import lax
from jax.experimental import pallas as pl
from jax.experimental.pallas import tpu as pltpu
from jax.experimental.pallas import tpu_sc as plsc

_CUTOFF = 4.0
_NUM_BASIS = 16

_NC = 2
_NS = 16
_NW = _NC * _NS

_C = 2000


def _sc_edge_diff(pos_flat, ei_flat, n):
    e = ei_flat.shape[0] // 2
    assert e % _NW == 0
    ew = e // _NW
    assert ew % _C == 0

    mesh = plsc.VectorSubcoreMesh(core_axis_name="c", subcore_axis_name="s")
    out_type = (
        jax.ShapeDtypeStruct((e,), jnp.float32),
        jax.ShapeDtypeStruct((e,), jnp.float32),
        jax.ShapeDtypeStruct((e,), jnp.float32),
    )
    scratch = [
        pltpu.VMEM((n,), jnp.float32),
        pltpu.VMEM((_C,), jnp.int32),
        pltpu.VMEM((_C,), jnp.int32),
        pltpu.VMEM((_C,), jnp.float32),
    ]

    @functools.partial(
        pl.kernel, out_type=out_type, mesh=mesh, scratch_types=scratch,
        compiler_params=pltpu.CompilerParams(needs_layout_passes=False))
    def sc_kernel(pos_hbm, ei_hbm, d0_hbm, d1_hbm, d2_hbm,
                  plane, sidx, didx, dbuf):
        wid = lax.axis_index("s") * _NC + lax.axis_index("c")

        for pofs, d_hbm in ((0, d0_hbm), (n, d1_hbm), (2 * n, d2_hbm)):
            pltpu.sync_copy(pos_hbm.at[pl.ds(pofs, n)], plane)

            def chunk(kk, _):
                cb = wid * ew + kk * _C
                pltpu.sync_copy(ei_hbm.at[pl.ds(cb, _C)], sidx)
                pltpu.sync_copy(ei_hbm.at[pl.ds(e + cb, _C)], didx)

                def step(i, _):
                    sl = pl.ds(i * 16, 16)
                    a = plsc.load_gather(plane, [didx[sl]])
                    b = plsc.load_gather(plane, [sidx[sl]])
                    dbuf[sl] = a - b
                    return 0

                lax.fori_loop(0, _C // 16, step, 0)
                pltpu.sync_copy(dbuf, d_hbm.at[pl.ds(cb, _C)])
                return 0

            lax.fori_loop(0, ew // _C, chunk, 0)

    return sc_kernel(pos_flat, ei_flat)


def _tc_node_embed(x, emb2t):
    n = x.shape[0]
    b = 3200
    assert n % b == 0
    nb = n // b
    xr = x.reshape(nb, 1, b)
    s = emb2t.shape[1]

    def body(x_ref, e_ref, hx_ref, hz_ref):
        xv = x_ref[...].reshape(1, b)
        sp = lax.broadcasted_iota(jnp.int32, (s, 1), 0)
        onehot = (xv == sp).astype(jnp.float32)
        t = lax.dot_general(e_ref[...], onehot, (((1,), (0,)), ((), ())),
                            preferred_element_type=jnp.float32)
        hx_ref[...] = t[:8, :]
        hz_ref[...] = t[8:, :]

    return pl.pallas_call(
        body,
        grid=(nb,),
        in_specs=[pl.BlockSpec((1, 1, b), lambda i: (i, 0, 0)),
                  pl.BlockSpec((16, s), lambda i: (0, 0))],
        out_specs=[pl.BlockSpec((8, b), lambda i: (0, i)),
                   pl.BlockSpec((8, b), lambda i: (0, i))],
        out_shape=[jax.ShapeDtypeStruct((8, n), jnp.float32),
                   jax.ShapeDtypeStruct((8, n), jnp.float32)],
    )(xr, emb2t)


_PI_A = 3.140625
_PI_B = 0.0009676535897932795
_S0 = 9.9999999372e-01
_S1 = -1.6666655189e-01
_S2 = 8.3329909945e-03
_S3 = -1.9805000098e-04
_S4 = 2.5966513689e-06


def _fast_sin(x):
    kf = jnp.floor(x * (1.0 / math.pi) + 0.5)
    y = (x - kf * _PI_A) - kf * _PI_B
    y2 = y * y
    p = y * (_S0 + y2 * (_S1 + y2 * (_S2 + y2 * (_S3 + y2 * _S4))))
    sbit = jnp.left_shift(jnp.bitwise_and(kf.astype(jnp.int32), 1), 31)
    bits = lax.bitcast_convert_type(p, jnp.int32) ^ sbit
    return lax.bitcast_convert_type(bits, jnp.float32)


def _tc_bessel(d0, d1, d2):
    e = d0.shape[0]
    b = 3200
    assert e % b == 0
    nb = e // b
    dr = [d.reshape(nb, 1, b) for d in (d0, d1, d2)]

    def body(xr, yr, zr, h_ref, a_ref):
        dx = xr[...].reshape(1, b)
        dy = yr[...].reshape(1, b)
        dz = zr[...].reshape(1, b)
        r = jnp.sqrt(dx * dx + dy * dy + dz * dz)
        g = math.sqrt(2.0 / _CUTOFF) / r
        nvec = (lax.broadcasted_iota(jnp.int32, (_NUM_BASIS, 1), 0) + 1
                ).astype(jnp.float32)
        theta = jnp.broadcast_to(r * (math.pi / _CUTOFF), (_NUM_BASIS, b))
        h_ref[...] = _fast_sin(theta * nvec) * g
        a_ref[...] = jnp.concatenate([dx, dy, dz], axis=0)

    return pl.pallas_call(
        body,
        grid=(nb,),
        in_specs=[pl.BlockSpec((1, 1, b), lambda i: (i, 0, 0))] * 3,
        out_specs=[pl.BlockSpec((_NUM_BASIS, b), lambda i: (0, i)),
                   pl.BlockSpec((3, b), lambda i: (0, i))],
        out_shape=[jax.ShapeDtypeStruct((_NUM_BASIS, e), jnp.float32),
                   jax.ShapeDtypeStruct((3, e), jnp.float32)],
    )(*dr)


def kernel(x, positions, edge_index, embed_node_x, embed_node_z):
    n = positions.shape[0]
    pos_flat = positions.T.reshape(-1)
    ei_flat = edge_index.astype(jnp.int32).reshape(-1)
    emb2t = jnp.concatenate([embed_node_x, embed_node_z], axis=1).T
    npad = -n % 3200
    xpad = jnp.pad(x.astype(jnp.int32), (0, npad))
    hx, hz = _tc_node_embed(xpad, emb2t)
    hx = lax.slice(hx, (0, 0), (8, n))
    hz = lax.slice(hz, (0, 0), (8, n))
    d0, d1, d2 = _sc_edge_diff(pos_flat, ei_flat, n)
    h16, a3 = _tc_bessel(d0, d1, d2)
    return (hx.T, hz.T, h16.T, a3.T)
